# baseline (speedup 1.0000x reference)
import jax
import jax.numpy as jnp
from jax.experimental import pallas as pl
from jax.experimental.pallas import tpu as pltpu

N_UNITS_ = 1000
U_BLK = 40
NBUF = 4


def _onehot_body(idxt_ref, out_hbm, scratch, sems):
    i = pl.program_id(0)
    nb = pl.num_programs(0)
    slot = jax.lax.rem(i, NBUF)

    def _copy(s, blk):
        return pltpu.make_async_copy(
            scratch.at[s],
            out_hbm.at[:, pl.ds(blk * U_BLK, U_BLK), :],
            sems.at[s],
        )

    @pl.when(i >= NBUF)
    def _():
        _copy(slot, i - NBUF).wait()

    u0 = i * U_BLK
    iota = u0 + jax.lax.broadcasted_iota(jnp.int32, scratch.shape[1:], 1)
    scratch[slot] = (idxt_ref[...][:, None, :] == iota).astype(jnp.float32)
    _copy(slot, i).start()

    @pl.when(i == nb - 1)
    def _():
        for k in range(NBUF):
            blk = nb - NBUF + k
            _copy(jax.lax.rem(blk, NBUF), blk).wait()


def kernel(indices):
    batch, n_active = indices.shape
    idx_t = indices.T
    oh_t = pl.pallas_call(
        _onehot_body,
        grid=(N_UNITS_ // U_BLK,),
        in_specs=[pl.BlockSpec((n_active, batch), lambda i: (0, 0))],
        out_specs=pl.BlockSpec(memory_space=pl.ANY),
        out_shape=jax.ShapeDtypeStruct((n_active, N_UNITS_, batch), jnp.float32),
        scratch_shapes=[
            pltpu.VMEM((NBUF, n_active, U_BLK, batch), jnp.float32),
            pltpu.SemaphoreType.DMA((NBUF,)),
        ],
    )(idx_t)
    return oh_t.transpose(2, 0, 1)
